# Initial kernel scaffold; baseline (speedup 1.0000x reference)
import jax
import jax.numpy as jnp
from jax.experimental import pallas as pl


def kernel(x, up_proj, down_proj, router_w, gate_w):
    raise NotImplementedError("write your pallas kernel here")


# speedup vs baseline: 34.1023x; 34.1023x over previous
import jax
import jax.numpy as jnp
from jax.experimental import pallas as pl


def _moe_kernel(x_ref, up_ref, down_ref, rw_ref, gw_ref, out_ref, ids_ref):
    xb = x_ref[...]
    logits = jax.lax.dot_general(
        xb, rw_ref[...], (((1,), (1,)), ((), ())),
        preferred_element_type=jnp.float32)
    ids = jnp.argmax(logits, axis=-1).astype(jnp.int32)

    up = jnp.dot(xb, up_ref[...], preferred_element_type=jnp.float32)
    act = up * jax.nn.sigmoid(up)

    B, EF = act.shape
    F = EF // rw_ref.shape[0]
    col_expert = jax.lax.broadcasted_iota(jnp.int32, (B, EF), 1) // F
    act = jnp.where(col_expert == ids[:, None], act, 0.0)

    expert_out = jnp.dot(act, down_ref[...], preferred_element_type=jnp.float32)

    gate_logit = jax.lax.dot_general(
        xb, gw_ref[...], (((1,), (1,)), ((), ())),
        preferred_element_type=jnp.float32)
    gate = jax.nn.sigmoid(gate_logit)

    out_ref[...] = xb + gate * expert_out
    ids_ref[0, 0, :] = ids


def kernel(x, up_proj, down_proj, router_w, gate_w):
    N, H = x.shape
    E, _, F = up_proj.shape
    up_all = up_proj.transpose(1, 0, 2).reshape(H, E * F)
    down_all = down_proj.reshape(E * F, H)

    BLK = 256
    grid = N // BLK
    out, ids3 = pl.pallas_call(
        _moe_kernel,
        grid=(grid,),
        in_specs=[
            pl.BlockSpec((BLK, H), lambda i: (i, 0)),
            pl.BlockSpec((H, E * F), lambda i: (0, 0)),
            pl.BlockSpec((E * F, H), lambda i: (0, 0)),
            pl.BlockSpec((E, H), lambda i: (0, 0)),
            pl.BlockSpec((1, H), lambda i: (0, 0)),
        ],
        out_specs=[
            pl.BlockSpec((BLK, H), lambda i: (i, 0)),
            pl.BlockSpec((1, 1, BLK), lambda i: (i, 0, 0)),
        ],
        out_shape=[
            jax.ShapeDtypeStruct((N, H), jnp.float32),
            jax.ShapeDtypeStruct((grid, 1, BLK), jnp.int32),
        ],
    )(x, up_all, down_all, router_w, gate_w)
    return out, ids3.reshape(N)


# speedup vs baseline: 36.5544x; 1.0719x over previous
import jax
import jax.numpy as jnp
from jax.experimental import pallas as pl
from jax.experimental.pallas import tpu as pltpu


def _moe_kernel(x_ref, up_ref, down_ref, rw_ref, gw_ref, out_ref, ids_ref,
                up_bf, down_bf):
    E, H, F = up_ref.shape

    @pl.when(pl.program_id(0) == 0)
    def _pack_weights():
        for e in range(E):
            up_bf[:, e * F:(e + 1) * F] = up_ref[e].astype(jnp.bfloat16)
            down_bf[e * F:(e + 1) * F, :] = down_ref[e].astype(jnp.bfloat16)

    xb = x_ref[...]
    logits = jax.lax.dot_general(
        xb, rw_ref[...], (((1,), (1,)), ((), ())),
        preferred_element_type=jnp.float32)
    ids = jnp.argmax(logits, axis=-1).astype(jnp.int32)

    xbf = xb.astype(jnp.bfloat16)
    up = jnp.dot(xbf, up_bf[...], preferred_element_type=jnp.float32)
    act = up * jax.nn.sigmoid(up)

    B, EF = act.shape
    col_expert = jax.lax.broadcasted_iota(jnp.int32, (B, EF), 1) // F
    act = jnp.where(col_expert == ids[:, None], act, 0.0)

    expert_out = jnp.dot(act.astype(jnp.bfloat16), down_bf[...],
                         preferred_element_type=jnp.float32)

    gate_logit = jax.lax.dot_general(
        xb, gw_ref[...], (((1,), (1,)), ((), ())),
        preferred_element_type=jnp.float32)
    gate = jax.nn.sigmoid(gate_logit)

    out_ref[...] = xb + gate * expert_out
    ids_ref[0, 0, :] = ids


def kernel(x, up_proj, down_proj, router_w, gate_w):
    N, H = x.shape
    E, _, F = up_proj.shape

    BLK = 256
    grid = N // BLK
    out, ids3 = pl.pallas_call(
        _moe_kernel,
        grid=(grid,),
        in_specs=[
            pl.BlockSpec((BLK, H), lambda i: (i, 0)),
            pl.BlockSpec((E, H, F), lambda i: (0, 0, 0)),
            pl.BlockSpec((E, F, H), lambda i: (0, 0, 0)),
            pl.BlockSpec((E, H), lambda i: (0, 0)),
            pl.BlockSpec((1, H), lambda i: (0, 0)),
        ],
        out_specs=[
            pl.BlockSpec((BLK, H), lambda i: (i, 0)),
            pl.BlockSpec((1, 1, BLK), lambda i: (i, 0, 0)),
        ],
        out_shape=[
            jax.ShapeDtypeStruct((N, H), jnp.float32),
            jax.ShapeDtypeStruct((grid, 1, BLK), jnp.int32),
        ],
        scratch_shapes=[
            pltpu.VMEM((H, E * F), jnp.bfloat16),
            pltpu.VMEM((E * F, H), jnp.bfloat16),
        ],
    )(x, up_proj, down_proj, router_w, gate_w)
    return out, ids3.reshape(N)


# speedup vs baseline: 37.8118x; 1.0344x over previous
import jax
import jax.numpy as jnp
from jax.experimental import pallas as pl
from jax.experimental.pallas import tpu as pltpu


def _moe_kernel(x_ref, up_ref, down_ref, rw_ref, gw_ref, out_ref, ids_ref,
                up_bf, down_bf):
    E, H, F = up_ref.shape

    @pl.when(pl.program_id(0) == 0)
    def _pack_weights():
        for e in range(E):
            up_bf[:, e * F:(e + 1) * F] = up_ref[e].astype(jnp.bfloat16)
            down_bf[e * F:(e + 1) * F, :] = down_ref[e].astype(jnp.bfloat16)

    xb = x_ref[...]
    logits = jax.lax.dot_general(
        xb, rw_ref[...], (((1,), (1,)), ((), ())),
        preferred_element_type=jnp.float32)
    ids = jnp.argmax(logits, axis=-1).astype(jnp.int32)

    xbf = xb.astype(jnp.bfloat16)
    up = jnp.dot(xbf, up_bf[...], preferred_element_type=jnp.float32)
    act = up * jax.nn.sigmoid(up)

    B, EF = act.shape
    col_expert = jax.lax.broadcasted_iota(jnp.int32, (B, EF), 1) // F
    act = jnp.where(col_expert == ids[:, None], act, 0.0)

    expert_out = jnp.dot(act.astype(jnp.bfloat16), down_bf[...],
                         preferred_element_type=jnp.float32)

    gate_logit = jax.lax.dot_general(
        xb, gw_ref[...], (((1,), (1,)), ((), ())),
        preferred_element_type=jnp.float32)
    gate = jax.nn.sigmoid(gate_logit)

    out_ref[...] = xb + gate * expert_out
    ids_ref[0, 0, :] = ids


def kernel(x, up_proj, down_proj, router_w, gate_w):
    N, H = x.shape
    E, _, F = up_proj.shape

    BLK = 512
    grid = N // BLK
    out, ids3 = pl.pallas_call(
        _moe_kernel,
        grid=(grid,),
        in_specs=[
            pl.BlockSpec((BLK, H), lambda i: (i, 0)),
            pl.BlockSpec((E, H, F), lambda i: (0, 0, 0)),
            pl.BlockSpec((E, F, H), lambda i: (0, 0, 0)),
            pl.BlockSpec((E, H), lambda i: (0, 0)),
            pl.BlockSpec((1, H), lambda i: (0, 0)),
        ],
        out_specs=[
            pl.BlockSpec((BLK, H), lambda i: (i, 0)),
            pl.BlockSpec((1, 1, BLK), lambda i: (i, 0, 0)),
        ],
        out_shape=[
            jax.ShapeDtypeStruct((N, H), jnp.float32),
            jax.ShapeDtypeStruct((grid, 1, BLK), jnp.int32),
        ],
        scratch_shapes=[
            pltpu.VMEM((H, E * F), jnp.bfloat16),
            pltpu.VMEM((E * F, H), jnp.bfloat16),
        ],
    )(x, up_proj, down_proj, router_w, gate_w)
    return out, ids3.reshape(N)
